# baseline (speedup 1.0000x reference)
import functools

import jax
import jax.numpy as jnp
from jax import lax
from jax.experimental import pallas as pl
from jax.experimental.pallas import tpu as pltpu
from jax.experimental.pallas import tpu_sc as plsc

NUM_SUMMONERS = 1000000
NUM_CHAMPIONS = 1000
NUM_FACTORS = 32
BATCH = 16384
PACK = 128 // NUM_FACTORS

_INFO = plsc.get_sparse_core_info()
NC = _INFO.num_cores
NS = _INFO.num_subcores
L = _INFO.num_lanes
NW = NC * NS
B_PER_W = BATCH // NW
CHUNK = 128
NCHUNK = B_PER_W // CHUNK


def _body(sid_hbm, cid_hbm, stab_hbm, ctab_hbm, out_hbm,
          sidx_v, cidx_v, gidx_v, srows_v, ctab_v, out_v, sem_s, sem_c):
    wid = lax.axis_index("s") * NC + lax.axis_index("c")
    base = wid * B_PER_W

    pltpu.sync_copy(sid_hbm.at[wid], sidx_v)
    pltpu.sync_copy(cid_hbm.at[wid], cidx_v)

    def srow_chunk(i, carry):
        j = i // (CHUNK // L)
        o = (i % (CHUNK // L)) * L
        gidx_v[j, pl.ds(o, L)] = sidx_v[j, pl.ds(o, L)] >> 2
        return carry

    lax.fori_loop(0, B_PER_W // L, srow_chunk, 0, unroll=8)

    cp_c = pltpu.async_copy(ctab_hbm, ctab_v, sem_c)
    copies = []
    for j in range(NCHUNK):
        dst = pl.ds(j * CHUNK, CHUNK)
        copies.append(
            pltpu.async_copy(stab_hbm.at[gidx_v.at[j]], srows_v.at[dst], sem_s))
    for cp in copies:
        cp.wait()
    cp_c.wait()

    iota = lax.iota(jnp.int32, L)

    def group(g, carry):
        j = g // (CHUNK // L)
        o = (g % (CHUNK // L)) * L
        sid = sidx_v[j, pl.ds(o, L)]
        cid = cidx_v[j, pl.ds(o, L)]
        scol0 = (sid & (PACK - 1)) * NUM_FACTORS
        crow = cid >> 2
        ccol0 = (cid & (PACK - 1)) * NUM_FACTORS
        rows = g * L + iota
        acc = jnp.zeros((L,), jnp.float32)
        for f in range(NUM_FACTORS):
            sv = plsc.load_gather(srows_v, [rows, scol0 + f])
            cv = plsc.load_gather(ctab_v, [crow, ccol0 + f])
            acc = acc + sv * cv
        out_v[pl.ds(g * L, L)] = acc
        return carry

    lax.fori_loop(0, B_PER_W // L, group, 0)

    pltpu.sync_copy(out_v, out_hbm.at[pl.ds(base, B_PER_W)])


@jax.jit
def kernel(summoner_ids, champ_ids, summoner_factors, champion_factors):
    sid = summoner_ids.astype(jnp.int32).reshape(NW, NCHUNK, CHUNK)
    cid = champ_ids.astype(jnp.int32).reshape(NW, NCHUNK, CHUNK)
    stab = summoner_factors.reshape(NUM_SUMMONERS // PACK, PACK * NUM_FACTORS)
    ctab = champion_factors.reshape(NUM_CHAMPIONS // PACK, PACK * NUM_FACTORS)
    mesh = plsc.VectorSubcoreMesh(core_axis_name="c", subcore_axis_name="s")
    run = pl.kernel(
        _body,
        out_type=jax.ShapeDtypeStruct((BATCH,), jnp.float32),
        mesh=mesh,
        compiler_params=pltpu.CompilerParams(
            needs_layout_passes=False, use_tc_tiling_on_sc=False),
        scratch_types=[
            pltpu.VMEM((NCHUNK, CHUNK), jnp.int32),
            pltpu.VMEM((NCHUNK, CHUNK), jnp.int32),
            pltpu.VMEM((NCHUNK, CHUNK), jnp.int32),
            pltpu.VMEM((B_PER_W, PACK * NUM_FACTORS), jnp.float32),
            pltpu.VMEM((NUM_CHAMPIONS // PACK, PACK * NUM_FACTORS), jnp.float32),
            pltpu.VMEM((B_PER_W,), jnp.float32),
            pltpu.SemaphoreType.DMA,
            pltpu.SemaphoreType.DMA,
        ],
    )
    return run(sid, cid, stab, ctab)


# speedup vs baseline: 1.0025x; 1.0025x over previous
import functools

import jax
import jax.numpy as jnp
from jax import lax
from jax.experimental import pallas as pl
from jax.experimental.pallas import tpu as pltpu
from jax.experimental.pallas import tpu_sc as plsc

NUM_SUMMONERS = 1000000
NUM_CHAMPIONS = 1000
NUM_FACTORS = 32
BATCH = 16384
PACK = 128 // NUM_FACTORS

_INFO = plsc.get_sparse_core_info()
NC = _INFO.num_cores
NS = _INFO.num_subcores
L = _INFO.num_lanes
NW = NC * NS
B_PER_W = BATCH // NW
CHUNK = 128
NCHUNK = B_PER_W // CHUNK


def _body(sid_hbm, cid_hbm, stab_hbm, ctab_hbm, out_hbm,
          sidx_v, cidx_v, gidx_v, srows_v, ctab_v, out_v, sem_s, sem_c):
    wid = lax.axis_index("s") * NC + lax.axis_index("c")
    base = wid * B_PER_W

    pltpu.sync_copy(sid_hbm.at[wid], sidx_v)
    pltpu.sync_copy(cid_hbm.at[wid], cidx_v)

    def srow_chunk(i, carry):
        j = i // (CHUNK // L)
        o = (i % (CHUNK // L)) * L
        gidx_v[j, pl.ds(o, L)] = sidx_v[j, pl.ds(o, L)] >> 2
        return carry

    lax.fori_loop(0, B_PER_W // L, srow_chunk, 0, unroll=8)

    cp_c = pltpu.async_copy(ctab_hbm, ctab_v, sem_c)
    copies = []
    for j in range(NCHUNK):
        dst = pl.ds(j * CHUNK, CHUNK)
        copies.append(
            pltpu.async_copy(stab_hbm.at[gidx_v.at[j]], srows_v.at[dst], sem_s))
    for cp in copies:
        cp.wait()
    cp_c.wait()

    iota = lax.iota(jnp.int32, L)

    def group(g, carry):
        j = g // (CHUNK // L)
        o = (g % (CHUNK // L)) * L
        sid = sidx_v[j, pl.ds(o, L)]
        cid = cidx_v[j, pl.ds(o, L)]
        scol0 = (sid & (PACK - 1)) * NUM_FACTORS
        crow = cid >> 2
        ccol0 = (cid & (PACK - 1)) * NUM_FACTORS
        rows = g * L + iota
        acc = jnp.zeros((L,), jnp.float32)
        for f in range(NUM_FACTORS):
            sv = plsc.load_gather(srows_v, [rows, scol0 + f])
            cv = plsc.load_gather(ctab_v, [crow, ccol0 + f])
            acc = acc + sv * cv
        out_v[pl.ds(g * L, L)] = acc
        return carry

    lax.fori_loop(0, B_PER_W // L, group, 0)

    pltpu.sync_copy(out_v, out_hbm.at[pl.ds(base, B_PER_W)])


@jax.jit
def kernel(summoner_ids, champ_ids, summoner_factors, champion_factors):
    sid = summoner_ids.astype(jnp.int32).reshape(NW, NCHUNK, CHUNK)
    cid = champ_ids.astype(jnp.int32).reshape(NW, NCHUNK, CHUNK)
    stab = summoner_factors.reshape(NUM_SUMMONERS // PACK, PACK * NUM_FACTORS)
    ctab = champion_factors.reshape(NUM_CHAMPIONS // PACK, PACK * NUM_FACTORS)
    mesh = plsc.VectorSubcoreMesh(core_axis_name="c", subcore_axis_name="s")
    run = pl.kernel(
        _body,
        out_type=jax.ShapeDtypeStruct((BATCH,), jnp.float32),
        mesh=mesh,
        compiler_params=pltpu.CompilerParams(
            needs_layout_passes=False, use_tc_tiling_on_sc=True),
        scratch_types=[
            pltpu.VMEM((NCHUNK, CHUNK), jnp.int32),
            pltpu.VMEM((NCHUNK, CHUNK), jnp.int32),
            pltpu.VMEM((NCHUNK, CHUNK), jnp.int32),
            pltpu.VMEM((B_PER_W, PACK * NUM_FACTORS), jnp.float32),
            pltpu.VMEM((NUM_CHAMPIONS // PACK, PACK * NUM_FACTORS), jnp.float32),
            pltpu.VMEM((B_PER_W,), jnp.float32),
            pltpu.SemaphoreType.DMA,
            pltpu.SemaphoreType.DMA,
        ],
    )
    return run(sid, cid, stab, ctab)


# speedup vs baseline: 1.0118x; 1.0093x over previous
import functools

import jax
import jax.numpy as jnp
from jax import lax
from jax.experimental import pallas as pl
from jax.experimental.pallas import tpu as pltpu
from jax.experimental.pallas import tpu_sc as plsc

NUM_SUMMONERS = 1000000
NUM_CHAMPIONS = 1000
NUM_FACTORS = 32
BATCH = 16384

_INFO = plsc.get_sparse_core_info()
NC = _INFO.num_cores
NS = _INFO.num_subcores
L = _INFO.num_lanes
NW = NC * NS
B_PER_W = BATCH // NW
CHUNK = 128
NCHUNK = B_PER_W // CHUNK


def _body(sid_hbm, cid_hbm, stab_hbm, ctab_hbm, out_hbm,
          sidx_v, cidx_v, srows_v, crows_v, out_v, sem_s, sem_c):
    wid = lax.axis_index("s") * NC + lax.axis_index("c")
    base = wid * B_PER_W

    pltpu.sync_copy(sid_hbm.at[pl.ds(wid * NCHUNK, NCHUNK)], sidx_v)
    pltpu.sync_copy(cid_hbm.at[pl.ds(wid * NCHUNK, NCHUNK)], cidx_v)

    copies = []
    for j in range(NCHUNK):
        dst = pl.ds(j * CHUNK, CHUNK)
        copies.append(
            pltpu.async_copy(stab_hbm.at[sidx_v.at[j]], srows_v.at[dst], sem_s))
        copies.append(
            pltpu.async_copy(ctab_hbm.at[cidx_v.at[j]], crows_v.at[dst], sem_c))
    for cp in copies:
        cp.wait()

    iota = lax.iota(jnp.int32, L)

    def group(g, carry):
        rows = g * L + iota
        acc = jnp.zeros((L,), jnp.float32)
        for f in range(NUM_FACTORS):
            col = jnp.full((L,), f, jnp.int32)
            sv = plsc.load_gather(srows_v, [rows, col])
            cv = plsc.load_gather(crows_v, [rows, col])
            acc = acc + sv * cv
        out_v[pl.ds(g * L, L)] = acc
        return carry

    lax.fori_loop(0, B_PER_W // L, group, 0)

    pltpu.sync_copy(out_v, out_hbm.at[pl.ds(base, B_PER_W)])


@jax.jit
def kernel(summoner_ids, champ_ids, summoner_factors, champion_factors):
    sid = summoner_ids.astype(jnp.int32).reshape(NW * NCHUNK, CHUNK)
    cid = champ_ids.astype(jnp.int32).reshape(NW * NCHUNK, CHUNK)
    mesh = plsc.VectorSubcoreMesh(core_axis_name="c", subcore_axis_name="s")
    run = pl.kernel(
        _body,
        out_type=jax.ShapeDtypeStruct((BATCH,), jnp.float32),
        mesh=mesh,
        compiler_params=pltpu.CompilerParams(
            needs_layout_passes=False, use_tc_tiling_on_sc=False),
        scratch_types=[
            pltpu.VMEM((NCHUNK, CHUNK), jnp.int32),
            pltpu.VMEM((NCHUNK, CHUNK), jnp.int32),
            pltpu.VMEM((B_PER_W, NUM_FACTORS), jnp.float32),
            pltpu.VMEM((B_PER_W, NUM_FACTORS), jnp.float32),
            pltpu.VMEM((B_PER_W,), jnp.float32),
            pltpu.SemaphoreType.DMA,
            pltpu.SemaphoreType.DMA,
        ],
    )
    return run(sid, cid, summoner_factors, champion_factors)
